# baseline (speedup 1.0000x reference)
import functools

import jax
import jax.numpy as jnp
from jax import lax
from jax.experimental import pallas as pl
from jax.experimental.pallas import tpu as pltpu
from jax.experimental.pallas import tpu_sc as plsc

N = 10000
NPAD = 10240
E = 320000
D_IN = 128
HID = 64
LAT = 32

NC = 2
NS = 16
NW = NC * NS
EPW = E // NW
CH = 80
NCHUNK = EPW // CH
RPT = NPAD // NS

_MESH = plsc.VectorSubcoreMesh(core_axis_name="c", subcore_axis_name="s")


def _make_sc_scatter(width):
    @functools.partial(
        pl.kernel,
        out_type=jax.ShapeDtypeStruct((NC, NPAD, width), jnp.float32),
        mesh=_MESH,
        compiler_params=pltpu.CompilerParams(use_tc_tiling_on_sc=False),
        scratch_types=[
            pltpu.VMEM((NCHUNK, CH), jnp.int32),
            pltpu.VMEM((NCHUNK, CH), jnp.int32),
            pltpu.VMEM((4, CH, width), jnp.float32),
            pltpu.VMEM_SHARED((NPAD, width), jnp.float32),
            [pltpu.SemaphoreType.DMA] * 4,
            [pltpu.SemaphoreType.DMA] * 4,
        ],
    )
    def sc_scatter(table_hbm, ei_hbm, zero_hbm, out_hbm,
                   src_v, dst_v, rows_v, accum, gsems, ssems):
        c = lax.axis_index("c")
        s = lax.axis_index("s")
        wid = c * NS + s
        pltpu.sync_copy(zero_hbm.at[pl.ds(s * RPT, RPT)],
                        accum.at[pl.ds(s * RPT, RPT)])
        pltpu.sync_copy(ei_hbm.at[0, wid], src_v)
        pltpu.sync_copy(ei_hbm.at[1, wid], dst_v)
        plsc.subcore_barrier()

        def gather(j, b):
            pltpu.async_copy(table_hbm.at[src_v.at[j]], rows_v.at[b],
                             gsems[b])

        def gwait(j, b):
            pltpu.make_async_copy(table_hbm.at[src_v.at[j]], rows_v.at[b],
                                  gsems[b]).wait()

        def scat(j, b):
            return pltpu.async_copy(rows_v.at[b], accum.at[dst_v.at[j]],
                                    ssems[b], add=True)

        def step(j, b):
            gwait(j, b)
            scat(j, b)
            b2 = (b + 2) % 4
            if isinstance(j, int) and j < 2:
                gather(j + 2, b2)
            else:
                pltpu.make_async_copy(rows_v.at[b2],
                                      accum.at[dst_v.at[j - 2]],
                                      ssems[b2]).wait()

                @pl.when(j + 2 < NCHUNK)
                def _refill():
                    gather(j + 2, b2)

        gather(0, 0)
        gather(1, 1)
        step(0, 0)
        step(1, 1)

        def body(g, _):
            for b in range(4):
                step(4 * g + 2 + b, (2 + b) % 4)
            return ()

        lax.fori_loop(0, (NCHUNK - 5) // 4, body, (), unroll=False)
        for j in range(NCHUNK - 3, NCHUNK):
            step(j, j % 4)
        for j in range(NCHUNK - 2, NCHUNK):
            b = j % 4
            pltpu.make_async_copy(rows_v.at[b], accum.at[dst_v.at[j]],
                                  ssems[b]).wait()
        plsc.subcore_barrier()
        pltpu.sync_copy(accum.at[pl.ds(s * RPT, RPT)],
                        out_hbm.at[c, pl.ds(s * RPT, RPT)])

    return sc_scatter


def _make_sc_degree():
    @functools.partial(
        pl.kernel,
        out_type=jax.ShapeDtypeStruct((NC, NPAD, 16), jnp.float32),
        mesh=_MESH,
        compiler_params=pltpu.CompilerParams(use_tc_tiling_on_sc=False),
        scratch_types=[
            pltpu.VMEM((NCHUNK, CH), jnp.int32),
            pltpu.VMEM((CH, 16), jnp.float32),
            pltpu.VMEM_SHARED((NPAD, 16), jnp.float32),
        ],
    )
    def sc_degree(ei_hbm, zero_hbm, out_hbm, dst_v, ones_v, accum):
        c = lax.axis_index("c")
        s = lax.axis_index("s")
        wid = c * NS + s
        pltpu.sync_copy(zero_hbm.at[pl.ds(s * RPT, RPT)],
                        accum.at[pl.ds(s * RPT, RPT)])
        pltpu.sync_copy(ei_hbm.at[1, wid], dst_v)

        def fill(i, _):
            ones_v[pl.ds(i * 16, 16), :] = jnp.ones((16, 16), jnp.float32)
            return ()

        lax.fori_loop(0, CH // 16, fill, (), unroll=True)
        plsc.subcore_barrier()

        def body(j, _):
            pltpu.sync_copy(ones_v, accum.at[dst_v.at[j]], add=True)
            return ()

        lax.fori_loop(0, NCHUNK, body, (), unroll=False)
        plsc.subcore_barrier()
        pltpu.sync_copy(accum.at[pl.ds(s * RPT, RPT)],
                        out_hbm.at[c, pl.ds(s * RPT, RPT)])

    return sc_degree


_sc_scatter64 = _make_sc_scatter(HID)
_sc_scatter32 = _make_sc_scatter(LAT)
_sc_degree = _make_sc_degree()

_BM = 2000


def _dis_of(dsum_block):
    return lax.rsqrt(dsum_block[:, 0:1] + 1.0)


def _tc_h0_body(x_ref, w_ref, o_ref):
    o_ref[...] = jnp.dot(x_ref[...], w_ref[...],
                         preferred_element_type=jnp.float32)


def _tc_scale_body(h_ref, dsum_ref, o_ref):
    o_ref[...] = h_ref[...] * _dis_of(dsum_ref[...])


def _tc_mid_body(u_ref, dsum_ref, b1_ref, g1_ref, be1_ref, w2_ref,
                 o_ref):
    dis = _dis_of(dsum_ref[...])
    t = u_ref[...] * dis + b1_ref[...]
    t = jnp.maximum(t, 0.0)
    mu = jnp.mean(t, axis=-1, keepdims=True)
    var = jnp.mean((t - mu) ** 2, axis=-1, keepdims=True)
    t = (t - mu) * lax.rsqrt(var + 1e-5) * g1_ref[...] + be1_ref[...]
    o_ref[...] = jnp.dot(t, w2_ref[...],
                         preferred_element_type=jnp.float32) * dis


def _tc_final_body(u_ref, dsum_ref, b2_ref, g2_ref, be2_ref,
                   wf1_ref, bf1_ref, wf2_ref, bf2_ref, lat_ref, rec_ref):
    dis = _dis_of(dsum_ref[...])
    t = u_ref[...] * dis + b2_ref[...]
    t = jnp.maximum(t, 0.0)
    mu = jnp.mean(t, axis=-1, keepdims=True)
    var = jnp.mean((t - mu) ** 2, axis=-1, keepdims=True)
    t = (t - mu) * lax.rsqrt(var + 1e-5) * g2_ref[...] + be2_ref[...]
    lat_ref[...] = t
    d = jnp.dot(t, wf1_ref[...], preferred_element_type=jnp.float32)
    d = jnp.maximum(d + bf1_ref[...], 0.0)
    rec_ref[...] = jnp.dot(d, wf2_ref[...],
                           preferred_element_type=jnp.float32) + bf2_ref[...]


def _row_spec(width):
    return pl.BlockSpec((_BM, width), lambda i: (i, 0))


def _rep_spec(shape):
    nd = len(shape)
    return pl.BlockSpec(shape, lambda i: (0,) * nd)


_tc_h0 = pl.pallas_call(
    _tc_h0_body,
    grid=(N // _BM,),
    in_specs=[_row_spec(D_IN), _rep_spec((D_IN, HID))],
    out_specs=_row_spec(HID),
    out_shape=jax.ShapeDtypeStruct((N, HID), jnp.float32),
)

_tc_scale = pl.pallas_call(
    _tc_scale_body,
    grid=(N // _BM,),
    in_specs=[_row_spec(HID), _row_spec(16)],
    out_specs=_row_spec(HID),
    out_shape=jax.ShapeDtypeStruct((N, HID), jnp.float32),
)

_tc_mid = pl.pallas_call(
    _tc_mid_body,
    grid=(N // _BM,),
    in_specs=[
        _row_spec(HID), _row_spec(16),
        _rep_spec((1, HID)), _rep_spec((1, HID)), _rep_spec((1, HID)),
        _rep_spec((HID, LAT)),
    ],
    out_specs=_row_spec(LAT),
    out_shape=jax.ShapeDtypeStruct((N, LAT), jnp.float32),
)

_tc_final = pl.pallas_call(
    _tc_final_body,
    grid=(N // _BM,),
    in_specs=[
        _row_spec(LAT), _row_spec(16),
        _rep_spec((1, LAT)), _rep_spec((1, LAT)), _rep_spec((1, LAT)),
        _rep_spec((LAT, HID)), _rep_spec((1, HID)),
        _rep_spec((HID, D_IN)), _rep_spec((1, D_IN)),
    ],
    out_specs=[_row_spec(LAT), _row_spec(D_IN)],
    out_shape=[
        jax.ShapeDtypeStruct((N, LAT), jnp.float32),
        jax.ShapeDtypeStruct((N, D_IN), jnp.float32),
    ],
)


def kernel(x, edge_index, W_gc1, b_gc1, g1, be1, W_gc2, b_gc2, g2, be2,
           W_fc1, b_fc1, W_fc2, b_fc2):
    ei = edge_index.reshape(2, NW, NCHUNK, CH)
    z16 = jnp.zeros((NPAD, 16), jnp.float32)
    z64 = jnp.zeros((NPAD, HID), jnp.float32)
    z32 = jnp.zeros((NPAD, LAT), jnp.float32)

    h1 = _tc_h0(x, W_gc1)
    degp = _sc_degree(ei, z16)
    dsum = degp[0] + degp[1]
    hs1 = _tc_scale(h1, dsum)
    p1 = _sc_scatter64(hs1, ei, z64)
    u1 = p1[0, :N] + p1[1, :N] + hs1
    hs2 = _tc_mid(u1, dsum, b_gc1[None, :], g1[None, :], be1[None, :],
                  W_gc2)
    p2 = _sc_scatter32(hs2, ei, z32)
    u2 = p2[0, :N] + p2[1, :N] + hs2
    latent, recon = _tc_final(u2, dsum, b_gc2[None, :], g2[None, :],
                              be2[None, :], W_fc1, b_fc1[None, :], W_fc2,
                              b_fc2[None, :])
    return (latent, recon)


# speedup vs baseline: 1.0266x; 1.0266x over previous
import functools

import jax
import jax.numpy as jnp
from jax import lax
from jax.experimental import pallas as pl
from jax.experimental.pallas import tpu as pltpu
from jax.experimental.pallas import tpu_sc as plsc

N = 10000
NPAD = 10240
E = 320000
D_IN = 128
HID = 64
LAT = 32

NC = 2
NS = 16
NW = NC * NS
EPW = E // NW
CH = 80
NCHUNK = EPW // CH
RPT = NPAD // NS

_MESH = plsc.VectorSubcoreMesh(core_axis_name="c", subcore_axis_name="s")


def _make_sc_scatter(width):
    @functools.partial(
        pl.kernel,
        out_type=jax.ShapeDtypeStruct((NC, NPAD, width), jnp.float32),
        mesh=_MESH,
        compiler_params=pltpu.CompilerParams(use_tc_tiling_on_sc=False),
        scratch_types=[
            pltpu.VMEM((NCHUNK, CH), jnp.int32),
            pltpu.VMEM((NCHUNK, CH), jnp.int32),
            pltpu.VMEM((4, CH, width), jnp.float32),
            pltpu.VMEM_SHARED((NPAD, width), jnp.float32),
            [pltpu.SemaphoreType.DMA] * 4,
            [pltpu.SemaphoreType.DMA] * 4,
        ],
    )
    def sc_scatter(table_hbm, ei_hbm, zero_hbm, out_hbm,
                   src_v, dst_v, rows_v, accum, gsems, ssems):
        c = lax.axis_index("c")
        s = lax.axis_index("s")
        wid = c * NS + s
        pltpu.sync_copy(zero_hbm.at[pl.ds(s * RPT, RPT)],
                        accum.at[pl.ds(s * RPT, RPT)])
        pltpu.sync_copy(ei_hbm.at[0, wid], src_v)
        pltpu.sync_copy(ei_hbm.at[1, wid], dst_v)
        plsc.subcore_barrier()

        def gather(j, b):
            pltpu.async_copy(table_hbm.at[src_v.at[j]], rows_v.at[b],
                             gsems[b])

        def gwait(j, b):
            pltpu.make_async_copy(table_hbm.at[src_v.at[j]], rows_v.at[b],
                                  gsems[b]).wait()

        def scat(j, b):
            return pltpu.async_copy(rows_v.at[b], accum.at[dst_v.at[j]],
                                    ssems[b], add=True)

        def step(j, b):
            gwait(j, b)
            scat(j, b)
            b2 = (b + 2) % 4
            if isinstance(j, int) and j < 2:
                gather(j + 2, b2)
            else:
                pltpu.make_async_copy(rows_v.at[b2],
                                      accum.at[dst_v.at[j - 2]],
                                      ssems[b2]).wait()

                @pl.when(j + 2 < NCHUNK)
                def _refill():
                    gather(j + 2, b2)

        gather(0, 0)
        gather(1, 1)
        step(0, 0)
        step(1, 1)

        def body(g, _):
            for b in range(4):
                step(4 * g + 2 + b, (2 + b) % 4)
            return ()

        lax.fori_loop(0, (NCHUNK - 5) // 4, body, (), unroll=False)
        for j in range(NCHUNK - 3, NCHUNK):
            step(j, j % 4)
        for j in range(NCHUNK - 2, NCHUNK):
            b = j % 4
            pltpu.make_async_copy(rows_v.at[b], accum.at[dst_v.at[j]],
                                  ssems[b]).wait()
        plsc.subcore_barrier()
        pltpu.sync_copy(accum.at[pl.ds(s * RPT, RPT)],
                        out_hbm.at[c, pl.ds(s * RPT, RPT)])

    return sc_scatter


def _make_sc_degree():
    @functools.partial(
        pl.kernel,
        out_type=jax.ShapeDtypeStruct((NC, NPAD, 16), jnp.float32),
        mesh=_MESH,
        compiler_params=pltpu.CompilerParams(use_tc_tiling_on_sc=False),
        scratch_types=[
            pltpu.VMEM((NCHUNK, CH), jnp.int32),
            pltpu.VMEM((CH, 16), jnp.float32),
            pltpu.VMEM_SHARED((NPAD, 16), jnp.float32),
            pltpu.SemaphoreType.DMA,
        ],
    )
    def sc_degree(ei_hbm, zero_hbm, out_hbm, dst_v, ones_v, accum, ssem):
        c = lax.axis_index("c")
        s = lax.axis_index("s")
        wid = c * NS + s
        pltpu.sync_copy(zero_hbm.at[pl.ds(s * RPT, RPT)],
                        accum.at[pl.ds(s * RPT, RPT)])
        pltpu.sync_copy(ei_hbm.at[1, wid], dst_v)

        def fill(i, _):
            ones_v[pl.ds(i * 16, 16), :] = jnp.ones((16, 16), jnp.float32)
            return ()

        lax.fori_loop(0, CH // 16, fill, (), unroll=True)
        plsc.subcore_barrier()

        def body(j, _):
            pltpu.async_copy(ones_v, accum.at[dst_v.at[j]], ssem, add=True)
            return ()

        lax.fori_loop(0, NCHUNK, body, (), unroll=False)

        def drain(j, _):
            pltpu.make_async_copy(ones_v, accum.at[dst_v.at[j]],
                                  ssem).wait()
            return ()

        lax.fori_loop(0, NCHUNK, drain, (), unroll=False)
        plsc.subcore_barrier()
        pltpu.sync_copy(accum.at[pl.ds(s * RPT, RPT)],
                        out_hbm.at[c, pl.ds(s * RPT, RPT)])

    return sc_degree


_sc_scatter64 = _make_sc_scatter(HID)
_sc_scatter32 = _make_sc_scatter(LAT)
_sc_degree = _make_sc_degree()

_BM = 2000


def _dis_of(dsum_block):
    return lax.rsqrt(dsum_block[:, 0:1] + 1.0)


def _tc_h0_body(x_ref, w_ref, o_ref):
    o_ref[...] = jnp.dot(x_ref[...], w_ref[...],
                         preferred_element_type=jnp.float32)


def _tc_scale_body(h_ref, dsum_ref, o_ref):
    o_ref[...] = h_ref[...] * _dis_of(dsum_ref[...])


def _tc_mid_body(u_ref, dsum_ref, b1_ref, g1_ref, be1_ref, w2_ref,
                 o_ref):
    dis = _dis_of(dsum_ref[...])
    t = u_ref[...] * dis + b1_ref[...]
    t = jnp.maximum(t, 0.0)
    mu = jnp.mean(t, axis=-1, keepdims=True)
    var = jnp.mean((t - mu) ** 2, axis=-1, keepdims=True)
    t = (t - mu) * lax.rsqrt(var + 1e-5) * g1_ref[...] + be1_ref[...]
    o_ref[...] = jnp.dot(t, w2_ref[...],
                         preferred_element_type=jnp.float32) * dis


def _tc_final_body(u_ref, dsum_ref, b2_ref, g2_ref, be2_ref,
                   wf1_ref, bf1_ref, wf2_ref, bf2_ref, lat_ref, rec_ref):
    dis = _dis_of(dsum_ref[...])
    t = u_ref[...] * dis + b2_ref[...]
    t = jnp.maximum(t, 0.0)
    mu = jnp.mean(t, axis=-1, keepdims=True)
    var = jnp.mean((t - mu) ** 2, axis=-1, keepdims=True)
    t = (t - mu) * lax.rsqrt(var + 1e-5) * g2_ref[...] + be2_ref[...]
    lat_ref[...] = t
    d = jnp.dot(t, wf1_ref[...], preferred_element_type=jnp.float32)
    d = jnp.maximum(d + bf1_ref[...], 0.0)
    rec_ref[...] = jnp.dot(d, wf2_ref[...],
                           preferred_element_type=jnp.float32) + bf2_ref[...]


def _row_spec(width):
    return pl.BlockSpec((_BM, width), lambda i: (i, 0))


def _rep_spec(shape):
    nd = len(shape)
    return pl.BlockSpec(shape, lambda i: (0,) * nd)


_tc_h0 = pl.pallas_call(
    _tc_h0_body,
    grid=(N // _BM,),
    in_specs=[_row_spec(D_IN), _rep_spec((D_IN, HID))],
    out_specs=_row_spec(HID),
    out_shape=jax.ShapeDtypeStruct((N, HID), jnp.float32),
)

_tc_scale = pl.pallas_call(
    _tc_scale_body,
    grid=(N // _BM,),
    in_specs=[_row_spec(HID), _row_spec(16)],
    out_specs=_row_spec(HID),
    out_shape=jax.ShapeDtypeStruct((N, HID), jnp.float32),
)

_tc_mid = pl.pallas_call(
    _tc_mid_body,
    grid=(N // _BM,),
    in_specs=[
        _row_spec(HID), _row_spec(16),
        _rep_spec((1, HID)), _rep_spec((1, HID)), _rep_spec((1, HID)),
        _rep_spec((HID, LAT)),
    ],
    out_specs=_row_spec(LAT),
    out_shape=jax.ShapeDtypeStruct((N, LAT), jnp.float32),
)

_tc_final = pl.pallas_call(
    _tc_final_body,
    grid=(N // _BM,),
    in_specs=[
        _row_spec(LAT), _row_spec(16),
        _rep_spec((1, LAT)), _rep_spec((1, LAT)), _rep_spec((1, LAT)),
        _rep_spec((LAT, HID)), _rep_spec((1, HID)),
        _rep_spec((HID, D_IN)), _rep_spec((1, D_IN)),
    ],
    out_specs=[_row_spec(LAT), _row_spec(D_IN)],
    out_shape=[
        jax.ShapeDtypeStruct((N, LAT), jnp.float32),
        jax.ShapeDtypeStruct((N, D_IN), jnp.float32),
    ],
)


def kernel(x, edge_index, W_gc1, b_gc1, g1, be1, W_gc2, b_gc2, g2, be2,
           W_fc1, b_fc1, W_fc2, b_fc2):
    ei = edge_index.reshape(2, NW, NCHUNK, CH)
    z16 = jnp.zeros((NPAD, 16), jnp.float32)
    z64 = jnp.zeros((NPAD, HID), jnp.float32)
    z32 = jnp.zeros((NPAD, LAT), jnp.float32)

    h1 = _tc_h0(x, W_gc1)
    degp = _sc_degree(ei, z16)
    dsum = degp[0] + degp[1]
    hs1 = _tc_scale(h1, dsum)
    p1 = _sc_scatter64(hs1, ei, z64)
    u1 = p1[0, :N] + p1[1, :N] + hs1
    hs2 = _tc_mid(u1, dsum, b_gc1[None, :], g1[None, :], be1[None, :],
                  W_gc2)
    p2 = _sc_scatter32(hs2, ei, z32)
    u2 = p2[0, :N] + p2[1, :N] + hs2
    latent, recon = _tc_final(u2, dsum, b_gc2[None, :], g2[None, :],
                              be2[None, :], W_fc1, b_fc1[None, :], W_fc2,
                              b_fc2[None, :])
    return (latent, recon)


# speedup vs baseline: 1.0451x; 1.0180x over previous
import functools

import jax
import jax.numpy as jnp
from jax import lax
from jax.experimental import pallas as pl
from jax.experimental.pallas import tpu as pltpu
from jax.experimental.pallas import tpu_sc as plsc

N = 10000
NPAD = 10240
E = 320000
D_IN = 128
HID = 64
LAT = 32

NC = 2
NS = 16
NW = NC * NS
EPW = E // NW
CH = 80
NCHUNK = EPW // CH
RPT = NPAD // NS

_MESH = plsc.VectorSubcoreMesh(core_axis_name="c", subcore_axis_name="s")


def _make_sc_scatter(width):
    @functools.partial(
        pl.kernel,
        out_type=jax.ShapeDtypeStruct((NC, NPAD, width), jnp.float32),
        mesh=_MESH,
        compiler_params=pltpu.CompilerParams(use_tc_tiling_on_sc=False),
        scratch_types=[
            pltpu.VMEM((NCHUNK, CH), jnp.int32),
            pltpu.VMEM((NCHUNK, CH), jnp.int32),
            pltpu.VMEM((4, CH, width), jnp.float32),
            pltpu.VMEM_SHARED((NPAD, width), jnp.float32),
            [pltpu.SemaphoreType.DMA] * 4,
            [pltpu.SemaphoreType.DMA] * 4,
        ],
    )
    def sc_scatter(table_hbm, ei_hbm, zero_hbm, out_hbm,
                   src_v, dst_v, rows_v, accum, gsems, ssems):
        c = lax.axis_index("c")
        s = lax.axis_index("s")
        wid = c * NS + s
        z = pltpu.async_copy(zero_hbm.at[pl.ds(s * RPT, RPT)],
                             accum.at[pl.ds(s * RPT, RPT)], gsems[3])
        sv = pltpu.async_copy(ei_hbm.at[0, wid], src_v, gsems[3])
        dv = pltpu.async_copy(ei_hbm.at[1, wid], dst_v, gsems[3])
        z.wait()
        sv.wait()
        dv.wait()
        plsc.subcore_barrier()

        def gather(j, b):
            pltpu.async_copy(table_hbm.at[src_v.at[j]], rows_v.at[b],
                             gsems[b])

        def gwait(j, b):
            pltpu.make_async_copy(table_hbm.at[src_v.at[j]], rows_v.at[b],
                                  gsems[b]).wait()

        def scat(j, b):
            return pltpu.async_copy(rows_v.at[b], accum.at[dst_v.at[j]],
                                    ssems[b], add=True)

        def step(j, b):
            gwait(j, b)
            scat(j, b)
            b2 = (b + 2) % 4
            if isinstance(j, int) and j < 2:
                gather(j + 2, b2)
            else:
                pltpu.make_async_copy(rows_v.at[b2],
                                      accum.at[dst_v.at[j - 2]],
                                      ssems[b2]).wait()

                @pl.when(j + 2 < NCHUNK)
                def _refill():
                    gather(j + 2, b2)

        gather(0, 0)
        gather(1, 1)
        step(0, 0)
        step(1, 1)

        def body(g, _):
            for b in range(4):
                step(4 * g + 2 + b, (2 + b) % 4)
            return ()

        lax.fori_loop(0, (NCHUNK - 5) // 4, body, (), unroll=False)
        for j in range(NCHUNK - 3, NCHUNK):
            step(j, j % 4)
        for j in range(NCHUNK - 2, NCHUNK):
            b = j % 4
            pltpu.make_async_copy(rows_v.at[b], accum.at[dst_v.at[j]],
                                  ssems[b]).wait()
        plsc.subcore_barrier()
        pltpu.sync_copy(accum.at[pl.ds(s * RPT, RPT)],
                        out_hbm.at[c, pl.ds(s * RPT, RPT)])

    return sc_scatter


def _make_sc_degree():
    @functools.partial(
        pl.kernel,
        out_type=jax.ShapeDtypeStruct((NC, NPAD, 16), jnp.float32),
        mesh=_MESH,
        compiler_params=pltpu.CompilerParams(use_tc_tiling_on_sc=False),
        scratch_types=[
            pltpu.VMEM((NCHUNK, CH), jnp.int32),
            pltpu.VMEM((CH, 16), jnp.float32),
            pltpu.VMEM_SHARED((NPAD, 16), jnp.float32),
            pltpu.SemaphoreType.DMA,
        ],
    )
    def sc_degree(ei_hbm, zero_hbm, out_hbm, dst_v, ones_v, accum, ssem):
        c = lax.axis_index("c")
        s = lax.axis_index("s")
        wid = c * NS + s
        z = pltpu.async_copy(zero_hbm.at[pl.ds(s * RPT, RPT)],
                             accum.at[pl.ds(s * RPT, RPT)], ssem)
        dv = pltpu.async_copy(ei_hbm.at[1, wid], dst_v, ssem)
        z.wait()
        dv.wait()

        def fill(i, _):
            ones_v[pl.ds(i * 16, 16), :] = jnp.ones((16, 16), jnp.float32)
            return ()

        lax.fori_loop(0, CH // 16, fill, (), unroll=True)
        plsc.subcore_barrier()

        def body(j, _):
            pltpu.async_copy(ones_v, accum.at[dst_v.at[j]], ssem, add=True)
            return ()

        lax.fori_loop(0, NCHUNK, body, (), unroll=False)

        def drain(j, _):
            pltpu.make_async_copy(ones_v, accum.at[dst_v.at[j]],
                                  ssem).wait()
            return ()

        lax.fori_loop(0, NCHUNK, drain, (), unroll=False)
        plsc.subcore_barrier()
        pltpu.sync_copy(accum.at[pl.ds(s * RPT, RPT)],
                        out_hbm.at[c, pl.ds(s * RPT, RPT)])

    return sc_degree


_sc_scatter64 = _make_sc_scatter(HID)
_sc_scatter32 = _make_sc_scatter(LAT)
_sc_degree = _make_sc_degree()

_BM = 2000


def _dis_of(dsum_block):
    return lax.rsqrt(dsum_block[:, 0:1] + 1.0)


def _tc_h0_body(x_ref, w_ref, o_ref):
    o_ref[...] = jnp.dot(x_ref[...], w_ref[...],
                         preferred_element_type=jnp.float32)


def _tc_scale_body(h_ref, dsum_ref, o_ref):
    o_ref[...] = h_ref[...] * _dis_of(dsum_ref[...])


def _tc_mid_body(u_ref, dsum_ref, b1_ref, g1_ref, be1_ref, w2_ref,
                 o_ref):
    dis = _dis_of(dsum_ref[...])
    t = u_ref[...] * dis + b1_ref[...]
    t = jnp.maximum(t, 0.0)
    mu = jnp.mean(t, axis=-1, keepdims=True)
    var = jnp.mean((t - mu) ** 2, axis=-1, keepdims=True)
    t = (t - mu) * lax.rsqrt(var + 1e-5) * g1_ref[...] + be1_ref[...]
    o_ref[...] = jnp.dot(t, w2_ref[...],
                         preferred_element_type=jnp.float32) * dis


def _tc_final_body(u_ref, dsum_ref, b2_ref, g2_ref, be2_ref,
                   wf1_ref, bf1_ref, wf2_ref, bf2_ref, lat_ref, rec_ref):
    dis = _dis_of(dsum_ref[...])
    t = u_ref[...] * dis + b2_ref[...]
    t = jnp.maximum(t, 0.0)
    mu = jnp.mean(t, axis=-1, keepdims=True)
    var = jnp.mean((t - mu) ** 2, axis=-1, keepdims=True)
    t = (t - mu) * lax.rsqrt(var + 1e-5) * g2_ref[...] + be2_ref[...]
    lat_ref[...] = t
    d = jnp.dot(t, wf1_ref[...], preferred_element_type=jnp.float32)
    d = jnp.maximum(d + bf1_ref[...], 0.0)
    rec_ref[...] = jnp.dot(d, wf2_ref[...],
                           preferred_element_type=jnp.float32) + bf2_ref[...]


def _row_spec(width):
    return pl.BlockSpec((_BM, width), lambda i: (i, 0))


def _rep_spec(shape):
    nd = len(shape)
    return pl.BlockSpec(shape, lambda i: (0,) * nd)


_tc_h0 = pl.pallas_call(
    _tc_h0_body,
    grid=(N // _BM,),
    in_specs=[_row_spec(D_IN), _rep_spec((D_IN, HID))],
    out_specs=_row_spec(HID),
    out_shape=jax.ShapeDtypeStruct((N, HID), jnp.float32),
)

_tc_scale = pl.pallas_call(
    _tc_scale_body,
    grid=(N // _BM,),
    in_specs=[_row_spec(HID), _row_spec(16)],
    out_specs=_row_spec(HID),
    out_shape=jax.ShapeDtypeStruct((N, HID), jnp.float32),
)

_tc_mid = pl.pallas_call(
    _tc_mid_body,
    grid=(N // _BM,),
    in_specs=[
        _row_spec(HID), _row_spec(16),
        _rep_spec((1, HID)), _rep_spec((1, HID)), _rep_spec((1, HID)),
        _rep_spec((HID, LAT)),
    ],
    out_specs=_row_spec(LAT),
    out_shape=jax.ShapeDtypeStruct((N, LAT), jnp.float32),
)

_tc_final = pl.pallas_call(
    _tc_final_body,
    grid=(N // _BM,),
    in_specs=[
        _row_spec(LAT), _row_spec(16),
        _rep_spec((1, LAT)), _rep_spec((1, LAT)), _rep_spec((1, LAT)),
        _rep_spec((LAT, HID)), _rep_spec((1, HID)),
        _rep_spec((HID, D_IN)), _rep_spec((1, D_IN)),
    ],
    out_specs=[_row_spec(LAT), _row_spec(D_IN)],
    out_shape=[
        jax.ShapeDtypeStruct((N, LAT), jnp.float32),
        jax.ShapeDtypeStruct((N, D_IN), jnp.float32),
    ],
)


def kernel(x, edge_index, W_gc1, b_gc1, g1, be1, W_gc2, b_gc2, g2, be2,
           W_fc1, b_fc1, W_fc2, b_fc2):
    ei = edge_index.reshape(2, NW, NCHUNK, CH)
    z16 = jnp.zeros((NPAD, 16), jnp.float32)
    z64 = jnp.zeros((NPAD, HID), jnp.float32)
    z32 = jnp.zeros((NPAD, LAT), jnp.float32)

    h1 = _tc_h0(x, W_gc1)
    degp = _sc_degree(ei, z16)
    dsum = degp[0] + degp[1]
    hs1 = _tc_scale(h1, dsum)
    p1 = _sc_scatter64(hs1, ei, z64)
    u1 = p1[0, :N] + p1[1, :N] + hs1
    hs2 = _tc_mid(u1, dsum, b_gc1[None, :], g1[None, :], be1[None, :],
                  W_gc2)
    p2 = _sc_scatter32(hs2, ei, z32)
    u2 = p2[0, :N] + p2[1, :N] + hs2
    latent, recon = _tc_final(u2, dsum, b_gc2[None, :], g2[None, :],
                              be2[None, :], W_fc1, b_fc1[None, :], W_fc2,
                              b_fc2[None, :])
    return (latent, recon)
